# Initial kernel scaffold; baseline (speedup 1.0000x reference)
import jax
import jax.numpy as jnp
from jax.experimental import pallas as pl


def kernel(x, W, b_dec):
    raise NotImplementedError("write your pallas kernel here")


# speedup vs baseline: 9.0434x; 9.0434x over previous
import functools

import jax
import jax.numpy as jnp
from jax.experimental import pallas as pl
from jax.experimental.pallas import tpu as pltpu

_K = 64
_ROW_TILE = 64


def _fused_body(x_ref, w_ref, b_ref, out_ref, feat_ref, *, k):
    xc = x_ref[...] - b_ref[...]
    feat = jax.lax.dot_general(
        xc, w_ref[...], (((1,), (0,)), ((), ())),
        preferred_element_type=jnp.float32)
    feat_ref[...] = feat

    rows = feat.shape[0]

    def bit_step(i, t):
        cand = t | (jnp.int32(1) << (jnp.int32(30) - i))
        cand_f = jax.lax.bitcast_convert_type(cand, jnp.float32)
        cnt = jnp.sum((jnp.abs(feat_ref[...]) >= cand_f).astype(jnp.int32),
                      axis=1, keepdims=True)
        return jnp.where(cnt >= k, cand, t)

    t = jax.lax.fori_loop(0, 31, bit_step, jnp.zeros((rows, 1), jnp.int32))
    t_f = jax.lax.bitcast_convert_type(t, jnp.float32)

    cur = feat_ref[...]
    masked = jnp.where(jnp.abs(cur) >= t_f, cur, 0.0)
    out_ref[...] = jax.lax.dot_general(
        masked, w_ref[...], (((1,), (1,)), ((), ())),
        preferred_element_type=jnp.float32) + b_ref[...]


@jax.jit
def kernel(x, W, b_dec):
    n, d = x.shape
    f = W.shape[1]
    row_tile = _ROW_TILE
    grid = (n // row_tile,)
    b2 = b_dec.reshape(1, d)
    return pl.pallas_call(
        functools.partial(_fused_body, k=_K),
        grid=grid,
        in_specs=[
            pl.BlockSpec((row_tile, d), lambda i: (i, 0)),
            pl.BlockSpec((d, f), lambda i: (0, 0)),
            pl.BlockSpec((1, d), lambda i: (0, 0)),
        ],
        out_specs=pl.BlockSpec((row_tile, d), lambda i: (i, 0)),
        out_shape=jax.ShapeDtypeStruct((n, d), jnp.float32),
        scratch_shapes=[
            pltpu.VMEM((row_tile, f), jnp.float32),
        ],
        compiler_params=pltpu.CompilerParams(
            dimension_semantics=("arbitrary",),
            vmem_limit_bytes=64 * 1024 * 1024,
        ),
    )(x, W, b2)


# speedup vs baseline: 11.1414x; 1.2320x over previous
import functools

import jax
import jax.numpy as jnp
from jax.experimental import pallas as pl
from jax.experimental.pallas import tpu as pltpu

_K = 64
_ROW_TILE = 64


def _fused_body(x_ref, w_ref, b_ref, out_ref, feat_ref, af_ref, *, k):
    xc = x_ref[...] - b_ref[...]
    feat = jax.lax.dot_general(
        xc, w_ref[...], (((1,), (0,)), ((), ())),
        preferred_element_type=jnp.float32)
    feat_ref[...] = feat
    af_ref[...] = jnp.abs(feat)

    rows = x_ref.shape[0]

    def _f(c):
        return jax.lax.bitcast_convert_type(c, jnp.float32)

    def bit_step(i, t):
        cand = t | (jnp.int32(1) << (jnp.int32(30) - i))
        cnt = jnp.sum((af_ref[...] >= _f(cand)).astype(jnp.float32), axis=1,
                      keepdims=True)
        return jnp.where(cnt >= float(k), cand, t)

    t = jax.lax.fori_loop(0, 31, bit_step, jnp.zeros((rows, 1), jnp.int32),
                          unroll=4)
    t_f = _f(t)

    masked = jnp.where(af_ref[...] >= t_f, feat_ref[...], 0.0)
    out_ref[...] = jax.lax.dot_general(
        masked, w_ref[...], (((1,), (1,)), ((), ())),
        preferred_element_type=jnp.float32) + b_ref[...]


@jax.jit
def kernel(x, W, b_dec):
    n, d = x.shape
    f = W.shape[1]
    row_tile = _ROW_TILE
    grid = (n // row_tile,)
    b2 = b_dec.reshape(1, d)
    return pl.pallas_call(
        functools.partial(_fused_body, k=_K),
        grid=grid,
        in_specs=[
            pl.BlockSpec((row_tile, d), lambda i: (i, 0)),
            pl.BlockSpec((d, f), lambda i: (0, 0)),
            pl.BlockSpec((1, d), lambda i: (0, 0)),
        ],
        out_specs=pl.BlockSpec((row_tile, d), lambda i: (i, 0)),
        out_shape=jax.ShapeDtypeStruct((n, d), jnp.float32),
        scratch_shapes=[
            pltpu.VMEM((row_tile, f), jnp.float32),
            pltpu.VMEM((row_tile, f), jnp.float32),
        ],
        compiler_params=pltpu.CompilerParams(
            dimension_semantics=("arbitrary",),
            vmem_limit_bytes=64 * 1024 * 1024,
        ),
    )(x, W, b2)
